# Initial kernel scaffold; baseline (speedup 1.0000x reference)
import jax
import jax.numpy as jnp
from jax.experimental import pallas as pl


def kernel(x, edge_index, W1, a_src1, a_dst1, b1, W2, a_src2, a_dst2, b2):
    raise NotImplementedError("write your pallas kernel here")


# speedup vs baseline: 26.7176x; 26.7176x over previous
import functools

import jax
import jax.numpy as jnp
from jax import lax
from jax.experimental import pallas as pl
from jax.experimental.pallas import tpu as pltpu
from jax.experimental.pallas import tpu_sc as plsc

N = 10000
E = 320000
D = 128
H1 = 4
C1 = 64
C2 = 128

NP = 10008
NA = 10112
ET = E + N
CHUNK = 64
ETP = 335872
ROWW = 144
BIG = 1e30


def _prep1_body(x_ref, w_ref, asm_ref, adm_ref, tbl_ref, adt_ref, mx_ref):
    xl = jnp.dot(x_ref[...], w_ref[...], preferred_element_type=jnp.float32)
    asrc = jnp.dot(xl, asm_ref[...], preferred_element_type=jnp.float32)
    adst = jnp.dot(xl, adm_ref[...], preferred_element_type=jnp.float32)
    tbl_ref[...] = jnp.zeros((2 * NP, ROWW), jnp.float32)
    adt_ref[...] = jnp.zeros((2 * NP, 16), jnp.float32)
    tbl_ref[0:N, 0:128] = xl[:, 0:128]
    tbl_ref[0:N, 128:130] = asrc[:, 0:2]
    tbl_ref[N:NP, 128:130] = jnp.full((NP - N, 2), -BIG, jnp.float32)
    tbl_ref[NP:NP + N, 0:128] = xl[:, 128:256]
    tbl_ref[NP:NP + N, 128:130] = asrc[:, 2:4]
    tbl_ref[NP + N:2 * NP, 128:130] = jnp.full((NP - N, 2), -BIG, jnp.float32)
    adt_ref[0:N, 0:2] = adst[:, 0:2]
    adt_ref[NP:NP + N, 0:2] = adst[:, 2:4]
    mx_ref[0:1, :] = jnp.max(asrc, axis=0, keepdims=True)
    mx_ref[1:2, :] = jnp.max(adst, axis=0, keepdims=True)


def _mid_body(acc_ref, b1_ref, w2_ref, a2s_ref, a2d_ref,
              tbl_ref, adt_ref, mx_ref):
    hcols = []
    for h in range(H1):
        base = (h // 2) * NA
        c0 = (h % 2) * 64
        num = acc_ref[base:base + N, c0:c0 + 64]
        den = acc_ref[base:base + N, 128 + (h % 2):129 + (h % 2)]
        hcols.append(num / den)
    hfeat = jnp.concatenate(hcols, axis=1) + b1_ref[...][None, :]
    hfeat = jnp.where(hfeat > 0, hfeat, jnp.exp(hfeat) - 1.0)
    xl2 = jnp.dot(hfeat, w2_ref[...], preferred_element_type=jnp.float32)
    asrc2 = jnp.dot(xl2, a2s_ref[...], preferred_element_type=jnp.float32)
    adst2 = jnp.dot(xl2, a2d_ref[...], preferred_element_type=jnp.float32)
    tbl_ref[...] = jnp.zeros((NP, ROWW), jnp.float32)
    adt_ref[...] = jnp.zeros((NP, 16), jnp.float32)
    tbl_ref[0:N, 0:128] = xl2
    tbl_ref[0:N, 128:129] = asrc2
    tbl_ref[N:NP, 128:129] = jnp.full((NP - N, 1), -BIG, jnp.float32)
    adt_ref[0:N, 0:1] = adst2
    mx_ref[0:1, 0:1] = jnp.max(asrc2, axis=0, keepdims=True)
    mx_ref[1:2, 0:1] = jnp.max(adst2, axis=0, keepdims=True)


def _final_body(acc_ref, b2_ref, out_ref):
    ssum = acc_ref[0:N, :] + acc_ref[NA:NA + N, :]
    out_ref[...] = ssum[:, 0:128] / ssum[:, 128:129] + b2_ref[...][None, :]


def _edge_pass(tbl, adt, mv, s_all, d_all, *, heads, split_cols):
    ept = ETP // 16 if split_cols else ETP // 32
    nch = ept // CHUNK
    groups_per_head = 8 // heads
    mesh = plsc.VectorSubcoreMesh(core_axis_name="c", subcore_axis_name="s")

    @functools.partial(
        pl.kernel,
        out_type=jax.ShapeDtypeStruct((2 * NA, ROWW), jnp.float32),
        mesh=mesh,
        compiler_params=pltpu.CompilerParams(use_tc_tiling_on_sc=False),
        scratch_types=[
            pltpu.VMEM((2, CHUNK), jnp.int32),
            pltpu.VMEM((2, CHUNK), jnp.int32),
            pltpu.VMEM((2, CHUNK), jnp.int32),
            pltpu.VMEM((2, CHUNK, ROWW), jnp.float32),
            pltpu.VMEM((CHUNK, ROWW), jnp.float32),
            pltpu.VMEM((2, CHUNK, 16), jnp.float32),
            pltpu.VMEM((16,), jnp.float32),
            pltpu.VMEM_SHARED((NA, ROWW), jnp.float32),
            pltpu.SemaphoreType.DMA,
            pltpu.SemaphoreType.DMA,
        ],
    )
    def k(tbl_h, adt_h, mv_h, s_h, d_h, out_h,
          s2, d2, d3, g, msg, adg, mvv, acc_sh, gs0, gs1):
        cid = lax.axis_index("c")
        sid = lax.axis_index("s")
        gsem = (gs0, gs1)

        zv = jnp.zeros((16,), jnp.float32)

        @pl.loop(0, CHUNK)
        def _(r):
            for j in range(ROWW // 16):
                msg[r, pl.ds(j * 16, 16)] = zv

        for i in range(9):
            pltpu.sync_copy(msg,
                            acc_sh.at[pl.ds(sid * 632 + i * CHUNK, CHUNK)])
        pltpu.sync_copy(msg.at[pl.ds(0, 56)],
                        acc_sh.at[pl.ds(sid * 632 + 9 * CHUNK, 56)])
        plsc.subcore_barrier()

        pltpu.sync_copy(mv_h.at[cid], mvv)
        base = (sid if split_cols else cid * 16 + sid) * ept

        def load_and_start(c, b):
            eb = base + c * CHUNK
            pltpu.sync_copy(s_h.at[pl.ds(eb, CHUNK)], s2.at[b])
            pltpu.sync_copy(d_h.at[pl.ds(eb, CHUNK)], d2.at[b])
            if split_cols:
                off = (cid * NP).astype(jnp.int32)
                for j in range(CHUNK // 16):
                    s2[b, pl.ds(j * 16, 16)] = s2[b, pl.ds(j * 16, 16)] + off
                    d3[b, pl.ds(j * 16, 16)] = d2[b, pl.ds(j * 16, 16)] + off
                idx = d3.at[b]
            else:
                idx = d2.at[b]
            pltpu.async_copy(tbl_h.at[s2.at[b]], g.at[b], gsem[b])
            pltpu.async_copy(adt_h.at[idx], adg.at[b], gsem[b])

        def handle(c, b):
            adt_idx = d3.at[b] if split_cols else d2.at[b]
            pltpu.make_async_copy(tbl_h.at[s2.at[b]], g.at[b],
                                  gsem[b]).wait()
            pltpu.make_async_copy(adt_h.at[adt_idx], adg.at[b],
                                  gsem[b]).wait()
            mv_vec = mvv[...]

            @plsc.parallel_loop(0, CHUNK, unroll=16)
            def _(kk):
                row = [g[b, kk, pl.ds(j * 16, 16)] for j in range(8)]
                av = g[b, kk, pl.ds(128, 16)] + adg[b, kk, :]
                lr = jnp.maximum(av, 0.0) + 0.2 * jnp.minimum(av, 0.0)
                ex = jnp.exp(lr - mv_vec)
                msg[kk, pl.ds(128, 16)] = ex
                ehb = [jnp.full((16,), ex[h], jnp.float32)
                       for h in range(heads)]
                out = [row[j] * ehb[j // groups_per_head] for j in range(8)]
                for j in range(8):
                    msg[kk, pl.ds(j * 16, 16)] = out[j]

            pltpu.sync_copy(msg, acc_sh.at[d2.at[b]], add=True)

            @pl.when(c + 2 < nch)
            def _():
                load_and_start(c + 2, b)

        load_and_start(0, 0)
        load_and_start(1, 1)

        @pl.loop(0, nch, step=2)
        def _(c):
            handle(c, 0)
            handle(c + 1, 1)

        plsc.subcore_barrier()
        pltpu.sync_copy(acc_sh.at[pl.ds(sid * 632, 632)],
                        out_h.at[pl.ds(cid * NA + sid * 632, 632)])

    return k(tbl, adt, mv, s_all, d_all)


def _tc_prep1(x, W1, asm, adm):
    return pl.pallas_call(
        _prep1_body,
        out_shape=[
            jax.ShapeDtypeStruct((2 * NP, ROWW), jnp.float32),
            jax.ShapeDtypeStruct((2 * NP, 16), jnp.float32),
            jax.ShapeDtypeStruct((2, H1), jnp.float32),
        ],
    )(x, W1, asm, adm)


def _tc_mid(acc1, b1, W2, a2s, a2d):
    return pl.pallas_call(
        _mid_body,
        out_shape=[
            jax.ShapeDtypeStruct((NP, ROWW), jnp.float32),
            jax.ShapeDtypeStruct((NP, 16), jnp.float32),
            jax.ShapeDtypeStruct((2, 1), jnp.float32),
        ],
    )(acc1, b1, W2, a2s, a2d)


def _tc_final(acc2, b2):
    return pl.pallas_call(
        _final_body,
        out_shape=jax.ShapeDtypeStruct((N, C2), jnp.float32),
    )(acc2, b2)


def _mv_rows(mx, nheads):
    m = mx[0] + mx[1]
    m = jnp.where(m > 0, m, 0.2 * m)
    if nheads == 4:
        mm = m.reshape(2, 2)
    else:
        mm = jnp.concatenate([m.reshape(1, 1), m.reshape(1, 1)], axis=0)
        mm = jnp.concatenate([mm, jnp.full((2, 1), BIG, jnp.float32)], axis=1)
    return jnp.concatenate([mm, jnp.full((2, 14), BIG, jnp.float32)], axis=1)


def kernel(x, edge_index, W1, a_src1, a_dst1, b1, W2, a_src2, a_dst2, b2):
    src = edge_index[0].astype(jnp.int32)
    dst = edge_index[1].astype(jnp.int32)
    loop = jnp.arange(N, dtype=jnp.int32)
    pad = ETP - ET
    s_all = jnp.concatenate([src, loop, jnp.full((pad,), N, jnp.int32)])
    d_all = jnp.concatenate([dst, loop, jnp.zeros((pad,), jnp.int32)])

    eye = jnp.eye(H1, dtype=jnp.float32)
    asm = (eye[:, None, :] * a_src1.reshape(H1, C1)[:, :, None]).reshape(
        H1 * C1, H1)
    adm = (eye[:, None, :] * a_dst1.reshape(H1, C1)[:, :, None]).reshape(
        H1 * C1, H1)
    a2s = a_src2.reshape(C2, 1)
    a2d = a_dst2.reshape(C2, 1)

    tbl1, adt1, mx1 = _tc_prep1(x, W1, asm, adm)
    mv1 = _mv_rows(mx1, 4)
    acc1 = _edge_pass(tbl1, adt1, mv1, s_all, d_all, heads=2, split_cols=True)
    tbl2, adt2, mx2 = _tc_mid(acc1, b1, W2, a2s, a2d)
    mv2 = _mv_rows(mx2, 1)
    acc2 = _edge_pass(tbl2, adt2, mv2, s_all, d_all, heads=1, split_cols=False)
    return _tc_final(acc2, b2)
